# baseline (speedup 1.0000x reference)
import functools
import math

import jax
import jax.numpy as jnp
from jax.experimental import pallas as pl
from jax.experimental.pallas import tpu as pltpu

_K = 16


def _rup(v, m):
    return (v + m - 1) // m * m


def _knn_kern(q_ref, d_ref, dn_ref, o_ref, *, k, nd, r, self_exclude):
    i = pl.program_id(0)
    qt = q_ref[...]
    dt = d_ref[...]
    qq = jnp.sum(qt * qt, axis=1, keepdims=True)
    g = jax.lax.dot_general(qt, dt, (((1,), (1,)), ((), ())),
                            preferred_element_type=jnp.float32)
    dist = jnp.maximum(qq + dn_ref[...] - 2.0 * g, 0.0)
    col = jax.lax.broadcasted_iota(jnp.int32, dist.shape, 1)
    inf = jnp.float32(jnp.inf)
    dist = jnp.where(col < nd, dist, inf)
    if self_exclude:
        row = i * r + jax.lax.broadcasted_iota(jnp.int32, dist.shape, 0)
        dist = jnp.where(col == row, inf, dist)
    ndp = dist.shape[1]
    out = jnp.zeros((r, 128), jnp.int32)
    col128 = jax.lax.broadcasted_iota(jnp.int32, (r, 128), 1)
    for t in range(k):
        m = jnp.min(dist, axis=1, keepdims=True)
        sel = jnp.min(jnp.where(dist == m, col, ndp), axis=1)
        out = jnp.where(col128 == t, sel[:, None], out)
        dist = jnp.where(col == sel[:, None], inf, dist)
    o_ref[...] = out


def _knn_pallas(qpos, dpos, k, self_exclude):
    nq, nd = qpos.shape[0], dpos.shape[0]
    r = 256 if nq >= 256 else 128
    nq_p = _rup(nq, r)
    nd_p = _rup(nd, 128)
    qp = jnp.zeros((nq_p, 8), jnp.float32).at[:nq, :3].set(qpos)
    dp = jnp.zeros((nd_p, 8), jnp.float32).at[:nd, :3].set(dpos)
    dn = jnp.sum(dp * dp, axis=1)[None, :]
    out = pl.pallas_call(
        functools.partial(_knn_kern, k=k, nd=nd, r=r,
                          self_exclude=self_exclude),
        grid=(nq_p // r,),
        in_specs=[
            pl.BlockSpec((r, 8), lambda i: (i, 0)),
            pl.BlockSpec((nd_p, 8), lambda i: (0, 0)),
            pl.BlockSpec((1, nd_p), lambda i: (0, 0)),
        ],
        out_specs=pl.BlockSpec((r, 128), lambda i: (i, 0)),
        out_shape=jax.ShapeDtypeStruct((nq_p, 128), jnp.int32),
    )(qp, dp, dn)
    return out[:nq, :k]


def _fps_kern(px_ref, py_ref, pz_ref, o_ref, d_ref, *, n, n_sub):
    px = px_ref[...]
    py = py_ref[...]
    pz = pz_ref[...]
    fi = (jax.lax.broadcasted_iota(jnp.int32, px.shape, 0) * 128
          + jax.lax.broadcasted_iota(jnp.int32, px.shape, 1))
    valid = fi < n
    inf = jnp.float32(jnp.inf)
    d_ref[...] = jnp.where(valid, inf, -inf)
    o_ref[0] = jnp.int32(0)

    def body(i, last):
        eq = fi == last
        z = jnp.float32(0.0)
        lx = jnp.sum(jnp.where(eq, px, z))
        ly = jnp.sum(jnp.where(eq, py, z))
        lz = jnp.sum(jnp.where(eq, pz, z))
        dx = px - lx
        dy = py - ly
        dz = pz - lz
        d = dx * dx + dy * dy + dz * dz
        nd_ = jnp.minimum(d_ref[...], d)
        d_ref[...] = nd_
        m = jnp.max(nd_)
        sel = jnp.min(jnp.where(nd_ == m, fi, n))
        o_ref[i] = sel
        return sel

    jax.lax.fori_loop(1, n_sub, body, jnp.int32(0))


def _fps_pallas(pos, n_sub):
    n = pos.shape[0]
    p = _rup(n, 128)
    s = p // 128
    cols = []
    for c in range(3):
        cols.append(jnp.zeros((p,), jnp.float32).at[:n].set(pos[:, c])
                    .reshape(s, 128))
    return pl.pallas_call(
        functools.partial(_fps_kern, n=n, n_sub=n_sub),
        in_specs=[pl.BlockSpec((s, 128), lambda: (0, 0))] * 3,
        out_specs=pl.BlockSpec(memory_space=pltpu.SMEM),
        out_shape=jax.ShapeDtypeStruct((n_sub,), jnp.int32),
        scratch_shapes=[pltpu.VMEM((s, 128), jnp.float32)],
    )(*cols)


def _mlp_bn(p, x):
    x = x @ p['lin']['W'] + p['lin']['b']
    mean = jnp.mean(x, 0)
    var = jnp.var(x, 0)
    x = (x - mean) / jnp.sqrt(var + 1e-5) * p['gamma'] + p['beta']
    return jax.nn.relu(x)


def _mlp2(p, x):
    x = jax.nn.relu(x @ p['l1']['W'] + p['l1']['b'])
    x = jax.nn.relu(x @ p['l2']['W'] + p['l2']['b'])
    return x


def _ptconv_d(p, x, pos, idx):
    q = x @ p['lin_q']['W']
    kk = x @ p['lin_k']['W']
    v = x @ p['lin_v']['W']
    delta = _mlp2(p['pos_nn'], pos[:, None, :] - pos[idx])
    alpha = _mlp2(p['attn_nn'], q[:, None, :] - kk[idx] + delta)
    amax = jnp.max(alpha, axis=1, keepdims=True)
    a = jnp.exp(alpha - amax)
    denom = jnp.sum(a, axis=1, keepdims=True)
    w = a / (denom + 1e-16)
    return jnp.sum(w * (v[idx] + delta), axis=1)


def _tblock_d(p, x, pos, idx):
    x = jax.nn.relu(x @ p['lin_in']['W'] + p['lin_in']['b'])
    x = _ptconv_d(p, x, pos, idx)
    x = jax.nn.relu(x @ p['lin_out']['W'] + p['lin_out']['b'])
    return x


def _transition_down(x, pos, mlp_p, k, ratio=0.25):
    n = pos.shape[0]
    n_sub = int(math.ceil(n * ratio))
    idx = _fps_pallas(pos, n_sub)
    sub_pos = pos[idx]
    nbr = _knn_pallas(sub_pos, pos, k, False)
    x = _mlp_bn(mlp_p, x)
    x_out = jnp.max(x[nbr], axis=1)
    return x_out, sub_pos


def _knn_interpolate(x_sub, pos_sub, pos, k=3):
    idx = _knn_pallas(pos, pos_sub, k, False)
    diff = pos[:, None, :] - pos_sub[idx]
    sqd = jnp.sum(diff * diff, -1)
    w = 1.0 / jnp.maximum(sqd, 1e-16)
    return jnp.sum(w[..., None] * x_sub[idx], 1) / jnp.sum(w, 1, keepdims=True)


def _transition_up(x_sub, pos_sub, p_sub, x, pos, p):
    x_sub = _mlp_bn(p_sub, x_sub)
    x_int = _knn_interpolate(x_sub, pos_sub, pos, 3)
    return _mlp_bn(p, x) + x_int, pos


def _forward(params, x, pos, k=_K):
    out_x, out_pos, graphs = [], [], []
    x = _mlp_bn(params['mlp_input'], x)
    out_x.append(x)
    out_pos.append(pos)
    n_lev = len(params['transformers_down'])
    for i in range(n_lev):
        graphs.append(_knn_pallas(pos, pos, k, True))
        x = _tblock_d(params['transformers_down'][i], x, pos, graphs[i])
        x, pos = _transition_down(x, pos, params['mlp_down'][i], k)
        out_x.append(x)
        out_pos.append(pos)
    graphs.append(_knn_pallas(pos, pos, k, True))
    x = _tblock_d(params['transformer'], x, pos, graphs[n_lev])
    x = _mlp_bn(params['mlp_summit'], x)
    for i in range(n_lev):
        lev = n_lev - i
        x = _tblock_d(params['transformers_up'][-i - 1], x, out_pos[lev],
                      graphs[lev])
        x, pos = _transition_up(x, pos, params['mlp_up_sub'][-i - 1],
                                out_x[-i - 2], out_pos[-i - 2],
                                params['mlp_up'][-i - 1])
    x = _tblock_d(params['end_transformer'], x, out_pos[0], graphs[0])
    h = jax.nn.relu(x @ params['lin1']['W'] + params['lin1']['b'])
    h = jax.nn.relu(h @ params['lin2']['W'] + params['lin2']['b'])
    out = h @ params['lin3']['W'] + params['lin3']['b']
    return jax.nn.log_softmax(out, -1)


def kernel(x, pos, batch, params):
    return _forward(params, x, pos)


# speedup vs baseline: 1.0231x; 1.0231x over previous
import functools
import math

import jax
import jax.numpy as jnp
from jax.experimental import pallas as pl
from jax.experimental.pallas import tpu as pltpu

_K = 16


def _rup(v, m):
    return (v + m - 1) // m * m


def _knn_kern(q_ref, d_ref, dn_ref, o_ref, *, k, nd, r, self_exclude):
    i = pl.program_id(0)
    qt = q_ref[...]
    dt = d_ref[...]
    qq = jnp.sum(qt * qt, axis=1, keepdims=True)
    g = jax.lax.dot_general(qt, dt, (((1,), (1,)), ((), ())),
                            preferred_element_type=jnp.float32)
    dist = jnp.maximum(qq + dn_ref[...] - 2.0 * g, 0.0)
    col = jax.lax.broadcasted_iota(jnp.int32, dist.shape, 1)
    inf = jnp.float32(jnp.inf)
    dist = jnp.where(col < nd, dist, inf)
    if self_exclude:
        row = i * r + jax.lax.broadcasted_iota(jnp.int32, dist.shape, 0)
        dist = jnp.where(col == row, inf, dist)
    ndp = dist.shape[1]
    out = jnp.zeros((r, 128), jnp.int32)
    col128 = jax.lax.broadcasted_iota(jnp.int32, (r, 128), 1)
    for t in range(k):
        m = jnp.min(dist, axis=1, keepdims=True)
        sel = jnp.min(jnp.where(dist == m, col, ndp), axis=1)
        out = jnp.where(col128 == t, sel[:, None], out)
        dist = jnp.where(col == sel[:, None], inf, dist)
    o_ref[...] = out


def _knn_pallas(qpos, dpos, k, self_exclude):
    nq, nd = qpos.shape[0], dpos.shape[0]
    r = 128
    nq_p = _rup(nq, r)
    nd_p = _rup(nd, 128)
    qp = jnp.zeros((nq_p, 8), jnp.float32).at[:nq, :3].set(qpos)
    dp = jnp.zeros((nd_p, 8), jnp.float32).at[:nd, :3].set(dpos)
    dn = jnp.sum(dp * dp, axis=1)[None, :]
    out = pl.pallas_call(
        functools.partial(_knn_kern, k=k, nd=nd, r=r,
                          self_exclude=self_exclude),
        grid=(nq_p // r,),
        in_specs=[
            pl.BlockSpec((r, 8), lambda i: (i, 0)),
            pl.BlockSpec((nd_p, 8), lambda i: (0, 0)),
            pl.BlockSpec((1, nd_p), lambda i: (0, 0)),
        ],
        out_specs=pl.BlockSpec((r, 128), lambda i: (i, 0)),
        out_shape=jax.ShapeDtypeStruct((nq_p, 128), jnp.int32),
    )(qp, dp, dn)
    return out[:nq, :k]


def _fps_kern(px_ref, py_ref, pz_ref, o_ref, d_ref, *, n, n_sub):
    px = px_ref[...]
    py = py_ref[...]
    pz = pz_ref[...]
    fi = (jax.lax.broadcasted_iota(jnp.int32, px.shape, 0) * 128
          + jax.lax.broadcasted_iota(jnp.int32, px.shape, 1))
    valid = fi < n
    inf = jnp.float32(jnp.inf)
    d_ref[...] = jnp.where(valid, inf, -inf)
    o_ref[0] = jnp.int32(0)

    def body(i, last):
        eq = fi == last
        z = jnp.float32(0.0)
        lx = jnp.sum(jnp.where(eq, px, z))
        ly = jnp.sum(jnp.where(eq, py, z))
        lz = jnp.sum(jnp.where(eq, pz, z))
        dx = px - lx
        dy = py - ly
        dz = pz - lz
        d = dx * dx + dy * dy + dz * dz
        nd_ = jnp.minimum(d_ref[...], d)
        d_ref[...] = nd_
        m = jnp.max(nd_)
        sel = jnp.min(jnp.where(nd_ == m, fi, n))
        o_ref[i] = sel
        return sel

    jax.lax.fori_loop(1, n_sub, body, jnp.int32(0))


def _fps_pallas(pos, n_sub):
    n = pos.shape[0]
    p = _rup(n, 128)
    s = p // 128
    cols = []
    for c in range(3):
        cols.append(jnp.zeros((p,), jnp.float32).at[:n].set(pos[:, c])
                    .reshape(s, 128))
    return pl.pallas_call(
        functools.partial(_fps_kern, n=n, n_sub=n_sub),
        in_specs=[pl.BlockSpec((s, 128), lambda: (0, 0))] * 3,
        out_specs=pl.BlockSpec(memory_space=pltpu.SMEM),
        out_shape=jax.ShapeDtypeStruct((n_sub,), jnp.int32),
        scratch_shapes=[pltpu.VMEM((s, 128), jnp.float32)],
    )(*cols)


def _mlp_bn(p, x):
    x = x @ p['lin']['W'] + p['lin']['b']
    mean = jnp.mean(x, 0)
    var = jnp.var(x, 0)
    x = (x - mean) / jnp.sqrt(var + 1e-5) * p['gamma'] + p['beta']
    return jax.nn.relu(x)


def _mlp2(p, x):
    x = jax.nn.relu(x @ p['l1']['W'] + p['l1']['b'])
    x = jax.nn.relu(x @ p['l2']['W'] + p['l2']['b'])
    return x


def _attn_kern(pd_ref, kg_ref, vg_ref, q_ref,
               w1_ref, b1_ref, w2_ref, b2_ref,
               a1_ref, c1_ref, a2_ref, c2_ref, o_ref, *, kfan):
    def mm(a, b):
        return jax.lax.dot_general(a, b, (((1,), (0,)), ((), ())),
                                   preferred_element_type=jnp.float32)
    pd = pd_ref[...]
    d1 = jnp.maximum(mm(pd, w1_ref[...]) + b1_ref[...], 0.0)
    delta = jnp.maximum(mm(d1, w2_ref[...]) + b2_ref[...], 0.0)
    rk, c = delta.shape
    r = rk // kfan
    q = q_ref[...]
    qrep = jnp.broadcast_to(q[:, None, :], (r, kfan, c)).reshape(rk, c)
    ain = qrep - kg_ref[...] + delta
    t1 = jnp.maximum(mm(ain, a1_ref[...]) + c1_ref[...], 0.0)
    alpha = jnp.maximum(mm(t1, a2_ref[...]) + c2_ref[...], 0.0)
    al3 = alpha.reshape(r, kfan, c)
    amax = jnp.max(al3, axis=1, keepdims=True)
    a = jnp.exp(al3 - amax)
    den = jnp.sum(a, axis=1, keepdims=True)
    w = a / (den + 1e-16)
    contrib = (vg_ref[...] + delta).reshape(r, kfan, c)
    o_ref[...] = jnp.sum(w * contrib, axis=1)


def _edge_attn(q, kg, vg, pd, pos_nn, attn_nn, kfan):
    n, c = q.shape
    h = pos_nn['l1']['W'].shape[1]
    r = 128 if c <= 128 else 32
    n_p = _rup(n, r)
    qp = jnp.zeros((n_p, c), jnp.float32).at[:n].set(q)
    pdp = jnp.zeros((n_p * kfan, 8), jnp.float32).at[:n * kfan, :3].set(pd)
    kgp = jnp.zeros((n_p * kfan, c), jnp.float32).at[:n * kfan].set(kg)
    vgp = jnp.zeros((n_p * kfan, c), jnp.float32).at[:n * kfan].set(vg)
    w1 = jnp.zeros((8, h), jnp.float32).at[:3].set(pos_nn['l1']['W'])
    out = pl.pallas_call(
        functools.partial(_attn_kern, kfan=kfan),
        grid=(n_p // r,),
        in_specs=[
            pl.BlockSpec((r * kfan, 8), lambda i: (i, 0)),
            pl.BlockSpec((r * kfan, c), lambda i: (i, 0)),
            pl.BlockSpec((r * kfan, c), lambda i: (i, 0)),
            pl.BlockSpec((r, c), lambda i: (i, 0)),
            pl.BlockSpec((8, h), lambda i: (0, 0)),
            pl.BlockSpec((1, h), lambda i: (0, 0)),
            pl.BlockSpec((h, c), lambda i: (0, 0)),
            pl.BlockSpec((1, c), lambda i: (0, 0)),
            pl.BlockSpec((c, h), lambda i: (0, 0)),
            pl.BlockSpec((1, h), lambda i: (0, 0)),
            pl.BlockSpec((h, c), lambda i: (0, 0)),
            pl.BlockSpec((1, c), lambda i: (0, 0)),
        ],
        out_specs=pl.BlockSpec((r, c), lambda i: (i, 0)),
        out_shape=jax.ShapeDtypeStruct((n_p, c), jnp.float32),
    )(pdp, kgp, vgp, qp,
      w1, pos_nn['l1']['b'][None, :], pos_nn['l2']['W'],
      pos_nn['l2']['b'][None, :],
      attn_nn['l1']['W'], attn_nn['l1']['b'][None, :],
      attn_nn['l2']['W'], attn_nn['l2']['b'][None, :])
    return out[:n]


def _ptconv_d(p, x, pos, idx):
    n, kfan = idx.shape
    q = x @ p['lin_q']['W']
    kk = x @ p['lin_k']['W']
    v = x @ p['lin_v']['W']
    c = q.shape[1]
    pd = (pos[:, None, :] - pos[idx]).reshape(n * kfan, 3)
    kg = kk[idx].reshape(n * kfan, c)
    vg = v[idx].reshape(n * kfan, c)
    return _edge_attn(q, kg, vg, pd, p['pos_nn'], p['attn_nn'], kfan)


def _tblock_d(p, x, pos, idx):
    x = jax.nn.relu(x @ p['lin_in']['W'] + p['lin_in']['b'])
    x = _ptconv_d(p, x, pos, idx)
    x = jax.nn.relu(x @ p['lin_out']['W'] + p['lin_out']['b'])
    return x


def _transition_down(x, pos, mlp_p, k, ratio=0.25):
    n = pos.shape[0]
    n_sub = int(math.ceil(n * ratio))
    idx = _fps_pallas(pos, n_sub)
    sub_pos = pos[idx]
    nbr = _knn_pallas(sub_pos, pos, k, False)
    x = _mlp_bn(mlp_p, x)
    x_out = jnp.max(x[nbr], axis=1)
    return x_out, sub_pos


def _knn_interpolate(x_sub, pos_sub, pos, k=3):
    idx = _knn_pallas(pos, pos_sub, k, False)
    diff = pos[:, None, :] - pos_sub[idx]
    sqd = jnp.sum(diff * diff, -1)
    w = 1.0 / jnp.maximum(sqd, 1e-16)
    return jnp.sum(w[..., None] * x_sub[idx], 1) / jnp.sum(w, 1, keepdims=True)


def _transition_up(x_sub, pos_sub, p_sub, x, pos, p):
    x_sub = _mlp_bn(p_sub, x_sub)
    x_int = _knn_interpolate(x_sub, pos_sub, pos, 3)
    return _mlp_bn(p, x) + x_int, pos


def _forward(params, x, pos, k=_K):
    out_x, out_pos, graphs = [], [], []
    x = _mlp_bn(params['mlp_input'], x)
    out_x.append(x)
    out_pos.append(pos)
    n_lev = len(params['transformers_down'])
    for i in range(n_lev):
        graphs.append(_knn_pallas(pos, pos, k, True))
        x = _tblock_d(params['transformers_down'][i], x, pos, graphs[i])
        x, pos = _transition_down(x, pos, params['mlp_down'][i], k)
        out_x.append(x)
        out_pos.append(pos)
    graphs.append(_knn_pallas(pos, pos, k, True))
    x = _tblock_d(params['transformer'], x, pos, graphs[n_lev])
    x = _mlp_bn(params['mlp_summit'], x)
    for i in range(n_lev):
        lev = n_lev - i
        x = _tblock_d(params['transformers_up'][-i - 1], x, out_pos[lev],
                      graphs[lev])
        x, pos = _transition_up(x, pos, params['mlp_up_sub'][-i - 1],
                                out_x[-i - 2], out_pos[-i - 2],
                                params['mlp_up'][-i - 1])
    x = _tblock_d(params['end_transformer'], x, out_pos[0], graphs[0])
    h = jax.nn.relu(x @ params['lin1']['W'] + params['lin1']['b'])
    h = jax.nn.relu(h @ params['lin2']['W'] + params['lin2']['b'])
    out = h @ params['lin3']['W'] + params['lin3']['b']
    return jax.nn.log_softmax(out, -1)


def kernel(x, pos, batch, params):
    return _forward(params, x, pos)


# speedup vs baseline: 1.1354x; 1.1098x over previous
import functools
import math

import jax
import jax.numpy as jnp
from jax.experimental import pallas as pl
from jax.experimental.pallas import tpu as pltpu

_K = 16


def _rup(v, m):
    return (v + m - 1) // m * m


def _knn_kern(q_ref, d_ref, dn_ref, o_ref, *, k, nd, r, self_exclude):
    i = pl.program_id(0)
    qt = q_ref[...]
    dt = d_ref[...]
    qq = jnp.sum(qt * qt, axis=1, keepdims=True)
    g = jax.lax.dot_general(qt, dt, (((1,), (1,)), ((), ())),
                            preferred_element_type=jnp.float32)
    dist = jnp.maximum(qq + dn_ref[...] - 2.0 * g, 0.0)
    col = jax.lax.broadcasted_iota(jnp.int32, dist.shape, 1)
    inf = jnp.float32(jnp.inf)
    dist = jnp.where(col < nd, dist, inf)
    if self_exclude:
        row = i * r + jax.lax.broadcasted_iota(jnp.int32, dist.shape, 0)
        dist = jnp.where(col == row, inf, dist)
    ndp = dist.shape[1]
    out = jnp.zeros((r, 128), jnp.int32)
    col128 = jax.lax.broadcasted_iota(jnp.int32, (r, 128), 1)
    for t in range(k):
        m = jnp.min(dist, axis=1, keepdims=True)
        sel = jnp.min(jnp.where(dist == m, col, ndp), axis=1)
        out = jnp.where(col128 == t, sel[:, None], out)
        dist = jnp.where(col == sel[:, None], inf, dist)
    o_ref[...] = out


def _knn_pallas(qpos, dpos, k, self_exclude):
    nq, nd = qpos.shape[0], dpos.shape[0]
    r = 128
    nq_p = _rup(nq, r)
    nd_p = _rup(nd, 128)
    qp = jnp.zeros((nq_p, 8), jnp.float32).at[:nq, :3].set(qpos)
    dp = jnp.zeros((nd_p, 8), jnp.float32).at[:nd, :3].set(dpos)
    dn = jnp.sum(dp * dp, axis=1)[None, :]
    out = pl.pallas_call(
        functools.partial(_knn_kern, k=k, nd=nd, r=r,
                          self_exclude=self_exclude),
        grid=(nq_p // r,),
        in_specs=[
            pl.BlockSpec((r, 8), lambda i: (i, 0)),
            pl.BlockSpec((nd_p, 8), lambda i: (0, 0)),
            pl.BlockSpec((1, nd_p), lambda i: (0, 0)),
        ],
        out_specs=pl.BlockSpec((r, 128), lambda i: (i, 0)),
        out_shape=jax.ShapeDtypeStruct((nq_p, 128), jnp.int32),
    )(qp, dp, dn)
    return out[:nq, :k]


def _fps_kern(px_ref, py_ref, pz_ref, o_ref, d_ref, *, n, n_sub):
    px = px_ref[...]
    py = py_ref[...]
    pz = pz_ref[...]
    fi = (jax.lax.broadcasted_iota(jnp.int32, px.shape, 0) * 128
          + jax.lax.broadcasted_iota(jnp.int32, px.shape, 1))
    valid = fi < n
    inf = jnp.float32(jnp.inf)
    d_ref[...] = jnp.where(valid, inf, -inf)
    o_ref[0] = jnp.int32(0)

    def body(i, last):
        eq = fi == last
        z = jnp.float32(0.0)
        lx = jnp.sum(jnp.where(eq, px, z))
        ly = jnp.sum(jnp.where(eq, py, z))
        lz = jnp.sum(jnp.where(eq, pz, z))
        dx = px - lx
        dy = py - ly
        dz = pz - lz
        d = dx * dx + dy * dy + dz * dz
        nd_ = jnp.minimum(d_ref[...], d)
        d_ref[...] = nd_
        m = jnp.max(nd_)
        sel = jnp.min(jnp.where(nd_ == m, fi, n))
        o_ref[i] = sel
        return sel

    jax.lax.fori_loop(1, n_sub, body, jnp.int32(0))


def _fps_pallas(pos, n_sub):
    n = pos.shape[0]
    p = _rup(n, 128)
    s = p // 128
    cols = []
    for c in range(3):
        cols.append(jnp.zeros((p,), jnp.float32).at[:n].set(pos[:, c])
                    .reshape(s, 128))
    return pl.pallas_call(
        functools.partial(_fps_kern, n=n, n_sub=n_sub),
        in_specs=[pl.BlockSpec((s, 128), lambda: (0, 0))] * 3,
        out_specs=pl.BlockSpec(memory_space=pltpu.SMEM),
        out_shape=jax.ShapeDtypeStruct((n_sub,), jnp.int32),
        scratch_shapes=[pltpu.VMEM((s, 128), jnp.float32)],
    )(*cols)


def _mlp_bn(p, x):
    x = x @ p['lin']['W'] + p['lin']['b']
    mean = jnp.mean(x, 0)
    var = jnp.var(x, 0)
    x = (x - mean) / jnp.sqrt(var + 1e-5) * p['gamma'] + p['beta']
    return jax.nn.relu(x)


def _mlp2(p, x):
    x = jax.nn.relu(x @ p['l1']['W'] + p['l1']['b'])
    x = jax.nn.relu(x @ p['l2']['W'] + p['l2']['b'])
    return x


def _attn_kern(pd_ref, kg_ref, vg_ref, q_ref,
               w1_ref, b1_ref, w2_ref, b2_ref,
               a1_ref, c1_ref, a2_ref, c2_ref, o_ref, *, kfan):
    def mm(a, b):
        return jax.lax.dot_general(a, b, (((1,), (0,)), ((), ())),
                                   preferred_element_type=jnp.float32)
    pd = pd_ref[...]
    d1 = jnp.maximum(mm(pd, w1_ref[...]) + b1_ref[...], 0.0)
    delta = jnp.maximum(mm(d1, w2_ref[...]) + b2_ref[...], 0.0)
    rk, c = delta.shape
    r = rk // kfan
    q = q_ref[...]
    qrep = jnp.broadcast_to(q[:, None, :], (r, kfan, c)).reshape(rk, c)
    ain = qrep - kg_ref[...] + delta
    t1 = jnp.maximum(mm(ain, a1_ref[...]) + c1_ref[...], 0.0)
    alpha = jnp.maximum(mm(t1, a2_ref[...]) + c2_ref[...], 0.0)
    al3 = alpha.reshape(r, kfan, c)
    amax = jnp.max(al3, axis=1, keepdims=True)
    a = jnp.exp(al3 - amax)
    den = jnp.sum(a, axis=1, keepdims=True)
    w = a / (den + 1e-16)
    contrib = (vg_ref[...] + delta).reshape(r, kfan, c)
    o_ref[...] = jnp.sum(w * contrib, axis=1)


def _edge_attn(q, kg, vg, pd, pos_nn, attn_nn, kfan):
    n, c = q.shape
    h = pos_nn['l1']['W'].shape[1]
    r = 128 if c <= 128 else 32
    n_p = _rup(n, r)
    qp = jnp.zeros((n_p, c), jnp.float32).at[:n].set(q)
    pdp = jnp.zeros((n_p * kfan, 8), jnp.float32).at[:n * kfan, :3].set(pd)
    kgp = jnp.zeros((n_p * kfan, c), jnp.float32).at[:n * kfan].set(kg)
    vgp = jnp.zeros((n_p * kfan, c), jnp.float32).at[:n * kfan].set(vg)
    w1 = jnp.zeros((8, h), jnp.float32).at[:3].set(pos_nn['l1']['W'])
    out = pl.pallas_call(
        functools.partial(_attn_kern, kfan=kfan),
        grid=(n_p // r,),
        in_specs=[
            pl.BlockSpec((r * kfan, 8), lambda i: (i, 0)),
            pl.BlockSpec((r * kfan, c), lambda i: (i, 0)),
            pl.BlockSpec((r * kfan, c), lambda i: (i, 0)),
            pl.BlockSpec((r, c), lambda i: (i, 0)),
            pl.BlockSpec((8, h), lambda i: (0, 0)),
            pl.BlockSpec((1, h), lambda i: (0, 0)),
            pl.BlockSpec((h, c), lambda i: (0, 0)),
            pl.BlockSpec((1, c), lambda i: (0, 0)),
            pl.BlockSpec((c, h), lambda i: (0, 0)),
            pl.BlockSpec((1, h), lambda i: (0, 0)),
            pl.BlockSpec((h, c), lambda i: (0, 0)),
            pl.BlockSpec((1, c), lambda i: (0, 0)),
        ],
        out_specs=pl.BlockSpec((r, c), lambda i: (i, 0)),
        out_shape=jax.ShapeDtypeStruct((n_p, c), jnp.float32),
    )(pdp, kgp, vgp, qp,
      w1, pos_nn['l1']['b'][None, :], pos_nn['l2']['W'],
      pos_nn['l2']['b'][None, :],
      attn_nn['l1']['W'], attn_nn['l1']['b'][None, :],
      attn_nn['l2']['W'], attn_nn['l2']['b'][None, :])
    return out[:n]


def _ptconv_d(p, x, pos, idx):
    n, kfan = idx.shape
    q = x @ p['lin_q']['W']
    kk = x @ p['lin_k']['W']
    v = x @ p['lin_v']['W']
    c = q.shape[1]
    g = jnp.concatenate([kk, v, pos], axis=1)[idx]
    kg = g[:, :, :c].reshape(n * kfan, c)
    vg = g[:, :, c:2 * c].reshape(n * kfan, c)
    pd = (pos[:, None, :] - g[:, :, 2 * c:]).reshape(n * kfan, 3)
    return _edge_attn(q, kg, vg, pd, p['pos_nn'], p['attn_nn'], kfan)


def _tblock_d(p, x, pos, idx):
    x = jax.nn.relu(x @ p['lin_in']['W'] + p['lin_in']['b'])
    x = _ptconv_d(p, x, pos, idx)
    x = jax.nn.relu(x @ p['lin_out']['W'] + p['lin_out']['b'])
    return x


def _transition_down(x, pos, mlp_p, k, ratio=0.25):
    n = pos.shape[0]
    n_sub = int(math.ceil(n * ratio))
    idx = _fps_pallas(pos, n_sub)
    sub_pos = pos[idx]
    nbr = _knn_pallas(sub_pos, pos, k, False)
    x = _mlp_bn(mlp_p, x)
    x_out = jnp.max(x[nbr], axis=1)
    return x_out, sub_pos


def _knn_interpolate(x_sub, pos_sub, pos, k=3):
    idx = _knn_pallas(pos, pos_sub, k, False)
    diff = pos[:, None, :] - pos_sub[idx]
    sqd = jnp.sum(diff * diff, -1)
    w = 1.0 / jnp.maximum(sqd, 1e-16)
    return jnp.sum(w[..., None] * x_sub[idx], 1) / jnp.sum(w, 1, keepdims=True)


def _transition_up(x_sub, pos_sub, p_sub, x, pos, p):
    x_sub = _mlp_bn(p_sub, x_sub)
    x_int = _knn_interpolate(x_sub, pos_sub, pos, 3)
    return _mlp_bn(p, x) + x_int, pos


def _forward(params, x, pos, k=_K):
    out_x, out_pos, graphs = [], [], []
    x = _mlp_bn(params['mlp_input'], x)
    out_x.append(x)
    out_pos.append(pos)
    n_lev = len(params['transformers_down'])
    for i in range(n_lev):
        graphs.append(_knn_pallas(pos, pos, k, True))
        x = _tblock_d(params['transformers_down'][i], x, pos, graphs[i])
        x, pos = _transition_down(x, pos, params['mlp_down'][i], k)
        out_x.append(x)
        out_pos.append(pos)
    graphs.append(_knn_pallas(pos, pos, k, True))
    x = _tblock_d(params['transformer'], x, pos, graphs[n_lev])
    x = _mlp_bn(params['mlp_summit'], x)
    for i in range(n_lev):
        lev = n_lev - i
        x = _tblock_d(params['transformers_up'][-i - 1], x, out_pos[lev],
                      graphs[lev])
        x, pos = _transition_up(x, pos, params['mlp_up_sub'][-i - 1],
                                out_x[-i - 2], out_pos[-i - 2],
                                params['mlp_up'][-i - 1])
    x = _tblock_d(params['end_transformer'], x, out_pos[0], graphs[0])
    h = jax.nn.relu(x @ params['lin1']['W'] + params['lin1']['b'])
    h = jax.nn.relu(h @ params['lin2']['W'] + params['lin2']['b'])
    out = h @ params['lin3']['W'] + params['lin3']['b']
    return jax.nn.log_softmax(out, -1)


def kernel(x, pos, batch, params):
    return _forward(params, x, pos)


# speedup vs baseline: 1.1442x; 1.0078x over previous
import functools
import math

import jax
import jax.numpy as jnp
from jax.experimental import pallas as pl
from jax.experimental.pallas import tpu as pltpu

_K = 16


def _rup(v, m):
    return (v + m - 1) // m * m


def _knn_kern(q_ref, d_ref, dn_ref, o_ref, *, k, nd, r, self_exclude):
    i = pl.program_id(0)
    qt = q_ref[...]
    dt = d_ref[...]
    qq = jnp.sum(qt * qt, axis=1, keepdims=True)
    g = jax.lax.dot_general(qt, dt, (((1,), (1,)), ((), ())),
                            preferred_element_type=jnp.float32)
    dist = jnp.maximum(qq + dn_ref[...] - 2.0 * g, 0.0)
    col = jax.lax.broadcasted_iota(jnp.int32, dist.shape, 1)
    inf = jnp.float32(jnp.inf)
    dist = jnp.where(col < nd, dist, inf)
    if self_exclude:
        row = i * r + jax.lax.broadcasted_iota(jnp.int32, dist.shape, 0)
        dist = jnp.where(col == row, inf, dist)
    ndp = dist.shape[1]
    out = jnp.zeros((r, 128), jnp.int32)
    col128 = jax.lax.broadcasted_iota(jnp.int32, (r, 128), 1)
    for t in range(k):
        m = jnp.min(dist, axis=1, keepdims=True)
        sel = jnp.min(jnp.where(dist == m, col, ndp), axis=1)
        out = jnp.where(col128 == t, sel[:, None], out)
        dist = jnp.where(col == sel[:, None], inf, dist)
    o_ref[...] = out


def _knn_pallas(qpos, dpos, k, self_exclude):
    nq, nd = qpos.shape[0], dpos.shape[0]
    r = 128
    nq_p = _rup(nq, r)
    nd_p = _rup(nd, 128)
    qp = jnp.zeros((nq_p, 8), jnp.float32).at[:nq, :3].set(qpos)
    dp = jnp.zeros((nd_p, 8), jnp.float32).at[:nd, :3].set(dpos)
    dn = jnp.sum(dp * dp, axis=1)[None, :]
    out = pl.pallas_call(
        functools.partial(_knn_kern, k=k, nd=nd, r=r,
                          self_exclude=self_exclude),
        grid=(nq_p // r,),
        in_specs=[
            pl.BlockSpec((r, 8), lambda i: (i, 0)),
            pl.BlockSpec((nd_p, 8), lambda i: (0, 0)),
            pl.BlockSpec((1, nd_p), lambda i: (0, 0)),
        ],
        out_specs=pl.BlockSpec((r, 128), lambda i: (i, 0)),
        out_shape=jax.ShapeDtypeStruct((nq_p, 128), jnp.int32),
    )(qp, dp, dn)
    return out[:nq, :k]


def _fps_kern(px_ref, py_ref, pz_ref, o_ref, d_ref, *, n, n_sub):
    px = px_ref[...]
    py = py_ref[...]
    pz = pz_ref[...]
    fi = (jax.lax.broadcasted_iota(jnp.int32, px.shape, 0) * 128
          + jax.lax.broadcasted_iota(jnp.int32, px.shape, 1))
    valid = fi < n
    inf = jnp.float32(jnp.inf)
    d_ref[...] = jnp.where(valid, inf, -inf)
    o_ref[0] = jnp.int32(0)

    def body(i, last):
        eq = fi == last
        z = jnp.float32(0.0)
        lx = jnp.sum(jnp.where(eq, px, z))
        ly = jnp.sum(jnp.where(eq, py, z))
        lz = jnp.sum(jnp.where(eq, pz, z))
        dx = px - lx
        dy = py - ly
        dz = pz - lz
        d = dx * dx + dy * dy + dz * dz
        nd_ = jnp.minimum(d_ref[...], d)
        d_ref[...] = nd_
        m = jnp.max(nd_)
        sel = jnp.min(jnp.where(nd_ == m, fi, n))
        o_ref[i] = sel
        return sel

    jax.lax.fori_loop(1, n_sub, body, jnp.int32(0))


def _fps_pallas(pos, n_sub):
    n = pos.shape[0]
    p = _rup(n, 128)
    s = p // 128
    cols = []
    for c in range(3):
        cols.append(jnp.zeros((p,), jnp.float32).at[:n].set(pos[:, c])
                    .reshape(s, 128))
    return pl.pallas_call(
        functools.partial(_fps_kern, n=n, n_sub=n_sub),
        in_specs=[pl.BlockSpec((s, 128), lambda: (0, 0))] * 3,
        out_specs=pl.BlockSpec(memory_space=pltpu.SMEM),
        out_shape=jax.ShapeDtypeStruct((n_sub,), jnp.int32),
        scratch_shapes=[pltpu.VMEM((s, 128), jnp.float32)],
    )(*cols)


def _mlp_bn(p, x):
    x = x @ p['lin']['W'] + p['lin']['b']
    mean = jnp.mean(x, 0)
    var = jnp.var(x, 0)
    x = (x - mean) / jnp.sqrt(var + 1e-5) * p['gamma'] + p['beta']
    return jax.nn.relu(x)


def _mlp2(p, x):
    x = jax.nn.relu(x @ p['l1']['W'] + p['l1']['b'])
    x = jax.nn.relu(x @ p['l2']['W'] + p['l2']['b'])
    return x


def _attn_kern(pd_ref, kg_ref, vg_ref, q_ref,
               w1_ref, b1_ref, w2_ref, b2_ref,
               a1_ref, c1_ref, a2_ref, c2_ref, o_ref, *, kfan):
    def mm(a, b):
        return jax.lax.dot_general(a, b, (((1,), (0,)), ((), ())),
                                   preferred_element_type=jnp.float32)
    pd = pd_ref[...]
    d1 = jnp.maximum(mm(pd, w1_ref[...]) + b1_ref[...], 0.0)
    delta = jnp.maximum(mm(d1, w2_ref[...]) + b2_ref[...], 0.0)
    rk, c = delta.shape
    r = rk // kfan
    q = q_ref[...]
    qrep = jnp.broadcast_to(q[:, None, :], (r, kfan, c)).reshape(rk, c)
    ain = qrep - kg_ref[...] + delta
    t1 = jnp.maximum(mm(ain, a1_ref[...]) + c1_ref[...], 0.0)
    alpha = jnp.maximum(mm(t1, a2_ref[...]) + c2_ref[...], 0.0)
    al3 = alpha.reshape(r, kfan, c)
    amax = jnp.max(al3, axis=1, keepdims=True)
    a = jnp.exp(al3 - amax)
    den = jnp.sum(a, axis=1, keepdims=True)
    w = a / (den + 1e-16)
    contrib = (vg_ref[...] + delta).reshape(r, kfan, c)
    o_ref[...] = jnp.sum(w * contrib, axis=1)


def _edge_attn(q, kg, vg, pd, pos_nn, attn_nn, kfan):
    n, c = q.shape
    h = pos_nn['l1']['W'].shape[1]
    r = 128 if c <= 128 else 32
    n_p = _rup(n, r)
    qp = jnp.zeros((n_p, c), jnp.float32).at[:n].set(q)
    pdp = jnp.zeros((n_p * kfan, 8), jnp.float32).at[:n * kfan, :3].set(pd)
    kgp = jnp.zeros((n_p * kfan, c), jnp.float32).at[:n * kfan].set(kg)
    vgp = jnp.zeros((n_p * kfan, c), jnp.float32).at[:n * kfan].set(vg)
    w1 = jnp.zeros((8, h), jnp.float32).at[:3].set(pos_nn['l1']['W'])
    out = pl.pallas_call(
        functools.partial(_attn_kern, kfan=kfan),
        grid=(n_p // r,),
        in_specs=[
            pl.BlockSpec((r * kfan, 8), lambda i: (i, 0)),
            pl.BlockSpec((r * kfan, c), lambda i: (i, 0)),
            pl.BlockSpec((r * kfan, c), lambda i: (i, 0)),
            pl.BlockSpec((r, c), lambda i: (i, 0)),
            pl.BlockSpec((8, h), lambda i: (0, 0)),
            pl.BlockSpec((1, h), lambda i: (0, 0)),
            pl.BlockSpec((h, c), lambda i: (0, 0)),
            pl.BlockSpec((1, c), lambda i: (0, 0)),
            pl.BlockSpec((c, h), lambda i: (0, 0)),
            pl.BlockSpec((1, h), lambda i: (0, 0)),
            pl.BlockSpec((h, c), lambda i: (0, 0)),
            pl.BlockSpec((1, c), lambda i: (0, 0)),
        ],
        out_specs=pl.BlockSpec((r, c), lambda i: (i, 0)),
        out_shape=jax.ShapeDtypeStruct((n_p, c), jnp.float32),
    )(pdp, kgp, vgp, qp,
      w1, pos_nn['l1']['b'][None, :], pos_nn['l2']['W'],
      pos_nn['l2']['b'][None, :],
      attn_nn['l1']['W'], attn_nn['l1']['b'][None, :],
      attn_nn['l2']['W'], attn_nn['l2']['b'][None, :])
    return out[:n]


def _ptconv_d(p, x, pos, idx):
    n, kfan = idx.shape
    q = x @ p['lin_q']['W']
    kk = x @ p['lin_k']['W']
    v = x @ p['lin_v']['W']
    c = q.shape[1]
    g = jnp.concatenate([kk, v, pos], axis=1)[idx]
    kg = g[:, :, :c].reshape(n * kfan, c)
    vg = g[:, :, c:2 * c].reshape(n * kfan, c)
    pd = (pos[:, None, :] - g[:, :, 2 * c:]).reshape(n * kfan, 3)
    return _edge_attn(q, kg, vg, pd, p['pos_nn'], p['attn_nn'], kfan)


def _tblock_d(p, x, pos, idx):
    x = jax.nn.relu(x @ p['lin_in']['W'] + p['lin_in']['b'])
    x = _ptconv_d(p, x, pos, idx)
    x = jax.nn.relu(x @ p['lin_out']['W'] + p['lin_out']['b'])
    return x


def _transition_down(x, pos, mlp_p, k, ratio=0.25):
    n = pos.shape[0]
    n_sub = int(math.ceil(n * ratio))
    idx = _fps_pallas(pos, n_sub)
    sub_pos = pos[idx]
    nbr = _knn_pallas(sub_pos, pos, k, False)
    x = _mlp_bn(mlp_p, x)
    x_out = jnp.max(x[nbr], axis=1)
    return x_out, sub_pos


def _knn_interpolate(x_sub, pos_sub, pos, k=3):
    idx = _knn_pallas(pos, pos_sub, k, False)
    c = x_sub.shape[1]
    g = jnp.concatenate([x_sub, pos_sub], axis=1)[idx]
    diff = pos[:, None, :] - g[:, :, c:]
    sqd = jnp.sum(diff * diff, -1)
    w = 1.0 / jnp.maximum(sqd, 1e-16)
    return jnp.sum(w[..., None] * g[:, :, :c], 1) / jnp.sum(w, 1, keepdims=True)


def _transition_up(x_sub, pos_sub, p_sub, x, pos, p):
    x_sub = _mlp_bn(p_sub, x_sub)
    x_int = _knn_interpolate(x_sub, pos_sub, pos, 3)
    return _mlp_bn(p, x) + x_int, pos


def _forward(params, x, pos, k=_K):
    out_x, out_pos, graphs = [], [], []
    x = _mlp_bn(params['mlp_input'], x)
    out_x.append(x)
    out_pos.append(pos)
    n_lev = len(params['transformers_down'])
    for i in range(n_lev):
        graphs.append(_knn_pallas(pos, pos, k, True))
        x = _tblock_d(params['transformers_down'][i], x, pos, graphs[i])
        x, pos = _transition_down(x, pos, params['mlp_down'][i], k)
        out_x.append(x)
        out_pos.append(pos)
    graphs.append(_knn_pallas(pos, pos, k, True))
    x = _tblock_d(params['transformer'], x, pos, graphs[n_lev])
    x = _mlp_bn(params['mlp_summit'], x)
    for i in range(n_lev):
        lev = n_lev - i
        x = _tblock_d(params['transformers_up'][-i - 1], x, out_pos[lev],
                      graphs[lev])
        x, pos = _transition_up(x, pos, params['mlp_up_sub'][-i - 1],
                                out_x[-i - 2], out_pos[-i - 2],
                                params['mlp_up'][-i - 1])
    x = _tblock_d(params['end_transformer'], x, out_pos[0], graphs[0])
    h = jax.nn.relu(x @ params['lin1']['W'] + params['lin1']['b'])
    h = jax.nn.relu(h @ params['lin2']['W'] + params['lin2']['b'])
    out = h @ params['lin3']['W'] + params['lin3']['b']
    return jax.nn.log_softmax(out, -1)


def kernel(x, pos, batch, params):
    return _forward(params, x, pos)
